# Initial kernel scaffold; baseline (speedup 1.0000x reference)
import jax
import jax.numpy as jnp
from jax.experimental import pallas as pl


def kernel(token_indices, aligned_embeddings):
    raise NotImplementedError("write your pallas kernel here")


# speedup vs baseline: 1.1279x; 1.1279x over previous
import functools

import jax
import jax.numpy as jnp
from jax import lax
from jax.experimental import pallas as pl
from jax.experimental.pallas import tpu as pltpu
from jax.experimental.pallas import tpu_sc as plsc

_B, _L, _D = 4096, 200, 64
_NW = 32
_RPW = _B // _NW
_NBUF = 4
_SPLIT = 104


def _encode_body(tok_hbm, table_hbm, out_hbm, idx_v, bufs, obuf, row0_v,
                 sem0, sem1, sem2, sem3):
    sems = (sem0, sem1, sem2, sem3)
    wid = lax.axis_index("s") * 2 + lax.axis_index("c")
    base = wid * _RPW

    pltpu.sync_copy(tok_hbm.at[pl.ds(base, _RPW)], idx_v)
    pltpu.sync_copy(table_hbm.at[pl.ds(0, 1)], row0_v)
    row0 = [row0_v[0, pl.ds(d * 16, 16)] for d in range(4)]
    lanes = lax.broadcasted_iota(jnp.int32, (16,), 0)

    def fire(r, b):
        buf = bufs.at[b]
        pltpu.async_copy(table_hbm.at[idx_v.at[r, pl.ds(0, _SPLIT)]],
                         buf.at[pl.ds(0, _SPLIT)], sems[b])
        pltpu.async_copy(table_hbm.at[idx_v.at[r, pl.ds(_SPLIT, _L - _SPLIT)]],
                         buf.at[pl.ds(_SPLIT, _L - _SPLIT)], sems[b])

    def process(r, b, prefetch_r):
        buf = bufs.at[b]
        pltpu.make_async_copy(table_hbm.at[pl.ds(0, _L)], buf, sems[b]).wait()

        nz = plsc.all_reduce_population_count(idx_v[r, pl.ds(0, 16)] == 0)
        for k in range(1, 12):
            nz = nz + plsc.all_reduce_population_count(
                idx_v[r, pl.ds(k * 16, 16)] == 0)
        tail = (idx_v[r, pl.ds(_L - 16, 16)] == 0) & (lanes >= 8)
        nz = nz + plsc.all_reduce_population_count(tail)

        zero = jnp.zeros((16,), jnp.float32)

        def acc_body(j, accs):
            accs = list(accs)
            rr = j * 8
            for u in range(8):
                for d in range(4):
                    slot = d * 2 + (u & 1)
                    accs[slot] = accs[slot] + buf[rr + u, pl.ds(d * 16, 16)]
            return tuple(accs)

        accs = lax.fori_loop(0, _L // 8, acc_body, (zero,) * 8)

        if prefetch_r is not None:
            fire(prefetch_r, b)

        nzf = nz.astype(jnp.float32)
        inv = 1.0 / (_L - nz).astype(jnp.float32)
        for d in range(4):
            res = (accs[d * 2] + accs[d * 2 + 1] - nzf * row0[d]) * inv
            obuf[r, pl.ds(d * 16, 16)] = res

    for b in range(_NBUF):
        fire(b, b)

    def outer(k, carry):
        for b in range(_NBUF):
            r = k * _NBUF + b
            process(r, b, r + _NBUF)
        return carry

    lax.fori_loop(0, _RPW // _NBUF - 1, outer, 0)
    for b in range(_NBUF):
        process(_RPW - _NBUF + b, b, None)

    pltpu.sync_copy(obuf, out_hbm.at[pl.ds(base, _RPW)])


_encoder = pl.kernel(
    _encode_body,
    out_type=jax.ShapeDtypeStruct((_B, _D), jnp.float32),
    mesh=plsc.VectorSubcoreMesh(core_axis_name="c", subcore_axis_name="s"),
    scratch_types=[
        pltpu.VMEM((_RPW, _L), jnp.int32),
        pltpu.VMEM((_NBUF, _L, _D), jnp.float32),
        pltpu.VMEM((_RPW, _D), jnp.float32),
        pltpu.VMEM((1, _D), jnp.float32),
        pltpu.SemaphoreType.DMA,
        pltpu.SemaphoreType.DMA,
        pltpu.SemaphoreType.DMA,
        pltpu.SemaphoreType.DMA,
    ],
    compiler_params=pltpu.CompilerParams(
        use_tc_tiling_on_sc=False, needs_layout_passes=False),
)


@jax.jit
def kernel(token_indices, aligned_embeddings):
    return _encoder(token_indices, aligned_embeddings)
